# baseline (speedup 1.0000x reference)
import functools

import jax
import jax.numpy as jnp
from jax import lax
from jax.experimental import pallas as pl
from jax.experimental.pallas import tpu as pltpu
from jax.experimental.pallas import tpu_sc as plsc

_N = 10000
_D = 128
_E = 320000
_NC = 2
_NS = 16
_NW = _NC * _NS
_EPW = _E // _NW
_C = 80
_NCH = _EPW // _C
_RPT = _N // _NS
_OPT = 624
_OTAIL = _N - _NS * _OPT


def _when(pred, fn):
    if isinstance(pred, (bool,)):
        if pred:
            fn()
    else:
        pl.when(pred)(fn)


def _spmm_body(x_hbm, src_hbm, dst_hbm, ew_hbm, y_hbm, yacc,
               sv0, sv1, sv2, sv3, dv0, dv1, dv2, dv3,
               ew0, ew1, ew2, ew3, rows_a, rows_b, rows_c,
               sem_ea, sem_eb, sem_ga, sem_gb, sem_gc,
               sem_sa, sem_sb, sem_sc):
    cid = lax.axis_index("c")
    sid = lax.axis_index("s")
    row0 = sid * _RPT
    ebase = (cid * _NS + sid) * _EPW

    srcv = (sv0, sv1, sv2, sv3)
    dstv = (dv0, dv1, dv2, dv3)
    eww = (ew0, ew1, ew2, ew3)
    rows = (rows_a, rows_b, rows_c)
    sem_e = (sem_ea, sem_eb)
    sem_g = (sem_ga, sem_gb, sem_gc)
    sem_s = (sem_sa, sem_sb, sem_sc)

    def _stage(k, j4):
        base = ebase + k * _C
        s = sem_e[j4 % 2]
        pltpu.async_copy(src_hbm.at[pl.ds(base, _C)], srcv[j4], s)
        pltpu.async_copy(dst_hbm.at[pl.ds(base, _C)], dstv[j4], s)
        pltpu.async_copy(ew_hbm.at[pl.ds(base, _C)], eww[j4], s)

    def _wait_stage(k, j4):
        base = ebase + k * _C
        s = sem_e[j4 % 2]
        pltpu.make_async_copy(src_hbm.at[pl.ds(base, _C)], srcv[j4], s).wait()
        pltpu.make_async_copy(dst_hbm.at[pl.ds(base, _C)], dstv[j4], s).wait()
        pltpu.make_async_copy(ew_hbm.at[pl.ds(base, _C)], eww[j4], s).wait()

    def _gather(j3, j4):
        pltpu.async_copy(x_hbm.at[srcv[j4]], rows[j3], sem_g[j3])

    def _wait_gather(j3, j4):
        pltpu.make_async_copy(x_hbm.at[srcv[j4]], rows[j3],
                              sem_g[j3]).wait()

    def _scatter(j3, j4):
        pltpu.async_copy(rows[j3], yacc.at[dstv[j4]], sem_s[j3], add=True)

    def _wait_scatter(j3, j4):
        pltpu.make_async_copy(rows[j3], yacc.at[dstv[j4]],
                              sem_s[j3]).wait()

    def _zrow(r, carry):
        for j in range(_D // 16):
            rows_a[r, pl.ds(j * 16, 16)] = jnp.zeros((16,), jnp.float32)
        return carry

    lax.fori_loop(0, _C, _zrow, 0)
    for cz in range(_RPT // _C):
        pltpu.sync_copy(rows_a, yacc.at[pl.ds(row0 + cz * _C, _C)])
    _zt = _RPT - (_RPT // _C) * _C
    if _zt:
        pltpu.sync_copy(rows_a.at[pl.ds(0, _zt)],
                        yacc.at[pl.ds(row0 + (_RPT // _C) * _C, _zt)])
    plsc.subcore_barrier()

    _stage(0, 0)
    _stage(1, 1)
    _wait_stage(0, 0)
    _gather(0, 0)

    def _scale(j3, j4):
        def _body(g, c2):
            wv = eww[j4][pl.ds(g * 16, 16)]
            for l in range(16):
                w = wv[l]
                rr = g * 16 + l
                for jj in range(_D // 16):
                    sl = pl.ds(jj * 16, 16)
                    rows[j3][rr, sl] = rows[j3][rr, sl] * w
            return c2

        lax.fori_loop(0, _C // 16, _body, 0)

    def _chunk(k, j3, j4):
        def _prep_next():
            _wait_stage(k + 1, (j4 + 1) % 4)

            def _free_rows():
                _wait_scatter((j3 + 1) % 3, (j4 + 2) % 4)

            _when(k >= 2, _free_rows)
            _gather((j3 + 1) % 3, (j4 + 1) % 4)

        _when(k + 1 < _NCH, _prep_next)
        _wait_gather(j3, j4)
        _scale(j3, j4)
        _scatter(j3, j4)

        def _next_stage():
            _stage(k + 2, (j4 + 2) % 4)

        _when(k + 2 < _NCH, _next_stage)

    def _twelve(i, carry):
        k = i * 12
        for j in range(12):
            _chunk(k + j, j % 3, j % 4)
        return carry

    lax.fori_loop(0, _NCH // 12, _twelve, 0)
    for k in range((_NCH // 12) * 12, _NCH):
        _chunk(k, k % 3, k % 4)
    _wait_scatter((_NCH - 2) % 3, (_NCH - 2) % 4)
    _wait_scatter((_NCH - 1) % 3, (_NCH - 1) % 4)
    plsc.subcore_barrier()

    out0 = sid * _OPT
    pltpu.sync_copy(yacc.at[pl.ds(out0, _OPT)],
                    y_hbm.at[cid, pl.ds(out0, _OPT)])

    @pl.when(sid == _NS - 1)
    def _tail():
        pltpu.sync_copy(yacc.at[pl.ds(_NS * _OPT, _OTAIL)],
                        y_hbm.at[cid, pl.ds(_NS * _OPT, _OTAIL)])


_spmm = functools.partial(
    pl.kernel,
    out_type=jax.ShapeDtypeStruct((_NC, _N, _D), jnp.float32),
    mesh=plsc.VectorSubcoreMesh(core_axis_name="c", subcore_axis_name="s"),
    scratch_types=[
        pltpu.VMEM_SHARED((_N, _D), jnp.float32),
        pltpu.VMEM((_C,), jnp.int32),
        pltpu.VMEM((_C,), jnp.int32),
        pltpu.VMEM((_C,), jnp.int32),
        pltpu.VMEM((_C,), jnp.int32),
        pltpu.VMEM((_C,), jnp.int32),
        pltpu.VMEM((_C,), jnp.int32),
        pltpu.VMEM((_C,), jnp.int32),
        pltpu.VMEM((_C,), jnp.int32),
        pltpu.VMEM((_C,), jnp.float32),
        pltpu.VMEM((_C,), jnp.float32),
        pltpu.VMEM((_C,), jnp.float32),
        pltpu.VMEM((_C,), jnp.float32),
        pltpu.VMEM((_C, _D), jnp.float32),
        pltpu.VMEM((_C, _D), jnp.float32),
        pltpu.VMEM((_C, _D), jnp.float32),
        pltpu.SemaphoreType.DMA,
        pltpu.SemaphoreType.DMA,
        pltpu.SemaphoreType.DMA,
        pltpu.SemaphoreType.DMA,
        pltpu.SemaphoreType.DMA,
        pltpu.SemaphoreType.DMA,
        pltpu.SemaphoreType.DMA,
        pltpu.SemaphoreType.DMA,
    ],
)(_spmm_body)


_BLK = 2000


def _mm1_body(x_ref, w_ref, b_ref, o_ref):
    kk = w_ref.shape[0]
    w1 = w_ref[0]
    for k in range(1, kk):
        w1 = w1 + w_ref[k]
    o_ref[...] = (jnp.dot(x_ref[...], w1, preferred_element_type=jnp.float32)
                  + b_ref[...])


def _mm1(x, W, b2):
    return pl.pallas_call(
        _mm1_body,
        grid=(_N // _BLK,),
        in_specs=[
            pl.BlockSpec((_BLK, _D), lambda i: (i, 0)),
            pl.BlockSpec((W.shape[0], _D, _D), lambda i: (0, 0, 0)),
            pl.BlockSpec((1, _D), lambda i: (0, 0)),
        ],
        out_specs=pl.BlockSpec((_BLK, _D), lambda i: (i, 0)),
        out_shape=jax.ShapeDtypeStruct((_N, _D), jnp.float32),
    )(x, W, b2)


def _mm2_body(o1_ref, y_ref, w_ref, o_ref):
    kk = w_ref.shape[0]
    w2 = jnp.zeros((_D, _D), jnp.float32)
    for k in range(1, kk):
        w2 = w2 + float(k) * w_ref[k]
    ys = y_ref[0] + y_ref[1]
    o_ref[...] = o1_ref[...] - jnp.dot(ys, w2,
                                       preferred_element_type=jnp.float32)


def _mm2(out1, y2, W):
    return pl.pallas_call(
        _mm2_body,
        grid=(_N // _BLK,),
        in_specs=[
            pl.BlockSpec((_BLK, _D), lambda i: (i, 0)),
            pl.BlockSpec((_NC, _BLK, _D), lambda i: (0, i, 0)),
            pl.BlockSpec((W.shape[0], _D, _D), lambda i: (0, 0, 0)),
        ],
        out_specs=pl.BlockSpec((_BLK, _D), lambda i: (i, 0)),
        out_shape=jax.ShapeDtypeStruct((_N, _D), jnp.float32),
    )(out1, y2, W)


def kernel(x, edge_index, edge_weight, W, b):
    y2 = _spmm(x, edge_index[0], edge_index[1], edge_weight)
    out1 = _mm1(x, W, b.reshape(1, _D))
    return _mm2(out1, y2, W)


# speedup vs baseline: 1.0561x; 1.0561x over previous
import functools

import jax
import jax.numpy as jnp
from jax import lax
from jax.experimental import pallas as pl
from jax.experimental.pallas import tpu as pltpu
from jax.experimental.pallas import tpu_sc as plsc

_N = 10000
_D = 128
_E = 320000
_NC = 2
_NS = 16
_NW = _NC * _NS
_EPW = _E // _NW
_C = 80
_NCH = _EPW // _C
_RPT = _N // _NS
_OPT = 624
_OTAIL = _N - _NS * _OPT


def _when(pred, fn):
    if isinstance(pred, (bool,)):
        if pred:
            fn()
    else:
        pl.when(pred)(fn)


def _spmm_body(x_hbm, src_hbm, dst_hbm, ew_hbm, y_hbm, yacc,
               sv0, sv1, sv2, sv3, dv0, dv1, dv2, dv3,
               ew0, ew1, ew2, ew3, rows_a, rows_b, rows_c,
               sem_ea, sem_eb, sem_ga, sem_gb, sem_gc,
               sem_sa, sem_sb, sem_sc):
    cid = lax.axis_index("c")
    sid = lax.axis_index("s")
    row0 = sid * _RPT
    ebase = (cid * _NS + sid) * _EPW

    srcv = (sv0, sv1, sv2, sv3)
    dstv = (dv0, dv1, dv2, dv3)
    eww = (ew0, ew1, ew2, ew3)
    rows = (rows_a, rows_b, rows_c)
    sem_e = (sem_ea, sem_eb)
    sem_g = (sem_ga, sem_gb, sem_gc)
    sem_s = (sem_sa, sem_sb, sem_sc)

    def _stage(k, j4):
        base = ebase + k * _C
        s = sem_e[j4 % 2]
        pltpu.async_copy(src_hbm.at[pl.ds(base, _C)], srcv[j4], s)
        pltpu.async_copy(dst_hbm.at[pl.ds(base, _C)], dstv[j4], s)
        pltpu.async_copy(ew_hbm.at[pl.ds(base, _C)], eww[j4], s)

    def _wait_stage(k, j4):
        base = ebase + k * _C
        s = sem_e[j4 % 2]
        pltpu.make_async_copy(src_hbm.at[pl.ds(base, _C)], srcv[j4], s).wait()
        pltpu.make_async_copy(dst_hbm.at[pl.ds(base, _C)], dstv[j4], s).wait()
        pltpu.make_async_copy(ew_hbm.at[pl.ds(base, _C)], eww[j4], s).wait()

    def _gather(j3, j4):
        pltpu.async_copy(x_hbm.at[srcv[j4]], rows[j3], sem_g[j3])

    def _wait_gather(j3, j4):
        pltpu.make_async_copy(x_hbm.at[srcv[j4]], rows[j3],
                              sem_g[j3]).wait()

    def _scatter(j3, j4):
        pltpu.async_copy(rows[j3], yacc.at[dstv[j4]], sem_s[j3], add=True)

    def _wait_scatter(j3, j4):
        pltpu.make_async_copy(rows[j3], yacc.at[dstv[j4]],
                              sem_s[j3]).wait()

    _stage(0, 0)
    _stage(1, 1)

    def _zrow(r, carry):
        for j in range(_D // 16):
            rows_c[r, pl.ds(j * 16, 16)] = jnp.zeros((16,), jnp.float32)
        return carry

    lax.fori_loop(0, _C, _zrow, 0)
    for cz in range(_RPT // _C):
        pltpu.async_copy(rows_c, yacc.at[pl.ds(row0 + cz * _C, _C)],
                         sem_sa)
    _zt = _RPT - (_RPT // _C) * _C
    if _zt:
        pltpu.async_copy(rows_c.at[pl.ds(0, _zt)],
                         yacc.at[pl.ds(row0 + (_RPT // _C) * _C, _zt)],
                         sem_sa)
    _wait_stage(0, 0)
    _gather(0, 0)
    for cz in range(_RPT // _C):
        pltpu.make_async_copy(rows_c, yacc.at[pl.ds(row0 + cz * _C, _C)],
                              sem_sa).wait()
    if _zt:
        pltpu.make_async_copy(
            rows_c.at[pl.ds(0, _zt)],
            yacc.at[pl.ds(row0 + (_RPT // _C) * _C, _zt)], sem_sa).wait()
    plsc.subcore_barrier()


    def _scale(j3, j4):
        def _body(g, c2):
            wv = eww[j4][pl.ds(g * 16, 16)]
            for l in range(16):
                w = jnp.full((16,), wv[l], jnp.float32)
                rr = g * 16 + l
                for jj in range(_D // 16):
                    sl = pl.ds(jj * 16, 16)
                    rows[j3][rr, sl] = rows[j3][rr, sl] * w
            return c2

        lax.fori_loop(0, _C // 16, _body, 0)

    def _chunk(k, j3, j4):
        def _prep_next():
            _wait_stage(k + 1, (j4 + 1) % 4)

            def _free_rows():
                _wait_scatter((j3 + 1) % 3, (j4 + 2) % 4)

            _when(k >= 2, _free_rows)
            _gather((j3 + 1) % 3, (j4 + 1) % 4)

        _when(k + 1 < _NCH, _prep_next)
        _wait_gather(j3, j4)
        _scale(j3, j4)
        _scatter(j3, j4)

        def _next_stage():
            _stage(k + 2, (j4 + 2) % 4)

        _when(k + 2 < _NCH, _next_stage)

    def _twelve(i, carry):
        k = i * 12
        for j in range(12):
            _chunk(k + j, j % 3, j % 4)
        return carry

    lax.fori_loop(0, _NCH // 12, _twelve, 0)
    for k in range((_NCH // 12) * 12, _NCH):
        _chunk(k, k % 3, k % 4)
    _wait_scatter((_NCH - 2) % 3, (_NCH - 2) % 4)
    _wait_scatter((_NCH - 1) % 3, (_NCH - 1) % 4)
    plsc.subcore_barrier()

    out0 = sid * _OPT
    pltpu.async_copy(yacc.at[pl.ds(out0, _OPT)],
                     y_hbm.at[cid, pl.ds(out0, _OPT)], sem_sa)

    @pl.when(sid == _NS - 1)
    def _tail():
        pltpu.async_copy(yacc.at[pl.ds(_NS * _OPT, _OTAIL)],
                         y_hbm.at[cid, pl.ds(_NS * _OPT, _OTAIL)], sem_sb)
        pltpu.make_async_copy(
            yacc.at[pl.ds(_NS * _OPT, _OTAIL)],
            y_hbm.at[cid, pl.ds(_NS * _OPT, _OTAIL)], sem_sb).wait()

    pltpu.make_async_copy(yacc.at[pl.ds(out0, _OPT)],
                          y_hbm.at[cid, pl.ds(out0, _OPT)], sem_sa).wait()


_spmm = functools.partial(
    pl.kernel,
    out_type=jax.ShapeDtypeStruct((_NC, _N, _D), jnp.float32),
    mesh=plsc.VectorSubcoreMesh(core_axis_name="c", subcore_axis_name="s"),
    scratch_types=[
        pltpu.VMEM_SHARED((_N, _D), jnp.float32),
        pltpu.VMEM((_C,), jnp.int32),
        pltpu.VMEM((_C,), jnp.int32),
        pltpu.VMEM((_C,), jnp.int32),
        pltpu.VMEM((_C,), jnp.int32),
        pltpu.VMEM((_C,), jnp.int32),
        pltpu.VMEM((_C,), jnp.int32),
        pltpu.VMEM((_C,), jnp.int32),
        pltpu.VMEM((_C,), jnp.int32),
        pltpu.VMEM((_C,), jnp.float32),
        pltpu.VMEM((_C,), jnp.float32),
        pltpu.VMEM((_C,), jnp.float32),
        pltpu.VMEM((_C,), jnp.float32),
        pltpu.VMEM((_C, _D), jnp.float32),
        pltpu.VMEM((_C, _D), jnp.float32),
        pltpu.VMEM((_C, _D), jnp.float32),
        pltpu.SemaphoreType.DMA,
        pltpu.SemaphoreType.DMA,
        pltpu.SemaphoreType.DMA,
        pltpu.SemaphoreType.DMA,
        pltpu.SemaphoreType.DMA,
        pltpu.SemaphoreType.DMA,
        pltpu.SemaphoreType.DMA,
        pltpu.SemaphoreType.DMA,
    ],
)(_spmm_body)


_BLK = 2000


def _mm1_body(x_ref, w_ref, b_ref, o_ref):
    kk = w_ref.shape[0]
    w1 = w_ref[0]
    for k in range(1, kk):
        w1 = w1 + w_ref[k]
    o_ref[...] = (jnp.dot(x_ref[...], w1, preferred_element_type=jnp.float32)
                  + b_ref[...])


def _mm1(x, W, b2):
    return pl.pallas_call(
        _mm1_body,
        grid=(_N // _BLK,),
        in_specs=[
            pl.BlockSpec((_BLK, _D), lambda i: (i, 0)),
            pl.BlockSpec((W.shape[0], _D, _D), lambda i: (0, 0, 0)),
            pl.BlockSpec((1, _D), lambda i: (0, 0)),
        ],
        out_specs=pl.BlockSpec((_BLK, _D), lambda i: (i, 0)),
        out_shape=jax.ShapeDtypeStruct((_N, _D), jnp.float32),
    )(x, W, b2)


def _mm2_body(o1_ref, y_ref, w_ref, o_ref):
    kk = w_ref.shape[0]
    w2 = jnp.zeros((_D, _D), jnp.float32)
    for k in range(1, kk):
        w2 = w2 + float(k) * w_ref[k]
    ys = y_ref[0] + y_ref[1]
    o_ref[...] = o1_ref[...] - jnp.dot(ys, w2,
                                       preferred_element_type=jnp.float32)


def _mm2(out1, y2, W):
    return pl.pallas_call(
        _mm2_body,
        grid=(_N // _BLK,),
        in_specs=[
            pl.BlockSpec((_BLK, _D), lambda i: (i, 0)),
            pl.BlockSpec((_NC, _BLK, _D), lambda i: (0, i, 0)),
            pl.BlockSpec((W.shape[0], _D, _D), lambda i: (0, 0, 0)),
        ],
        out_specs=pl.BlockSpec((_BLK, _D), lambda i: (i, 0)),
        out_shape=jax.ShapeDtypeStruct((_N, _D), jnp.float32),
    )(out1, y2, W)


def kernel(x, edge_index, edge_weight, W, b):
    y2 = _spmm(x, edge_index[0], edge_index[1], edge_weight)
    out1 = _mm1(x, W, b.reshape(1, _D))
    return _mm2(out1, y2, W)
